# Initial kernel scaffold; baseline (speedup 1.0000x reference)
import functools

import jax
import jax.numpy as jnp
from jax import lax
from jax.experimental import pallas as pl
from jax.experimental.pallas import tpu as pltpu
from jax.experimental.pallas import tpu_sc as plsc

N = 10000
E = 160000
F_IN = 128
H = 16
R = 8

NW = 32
CHUNK = 128
NCH = 40
EPW = NCH * CHUNK
EP = NW * EPW
NPT = N // 16
BLK = 2500


def _edge_prep_body(ew_ref, et_ref, src_ref, ewn_ref, idx_ref):
    ew = ew_ref[...]
    mn = jnp.min(ew)
    mx = jnp.max(ew)
    inv = 1.0 / (mx - mn + 1e-8)
    ewn_ref[...] = (ew - mn) * inv
    idx_ref[...] = src_ref[...] * R + et_ref[...].astype(jnp.int32)


def _dense1_body(x_ref, w_ref, root_ref, b_ref, y_ref, xr_ref):
    x = x_ref[...]
    y_ref[...] = jnp.dot(x, w_ref[...], preferred_element_type=jnp.float32)
    xr_ref[...] = (
        jnp.dot(x, root_ref[...], preferred_element_type=jnp.float32) + b_ref[...]
    )


def _mid_body(p0_ref, p1_ref, xr_ref, w_ref, root_ref, b_ref, y_ref, hr_ref):
    h = jnp.maximum(p0_ref[...] + p1_ref[...] + xr_ref[...], 0.0)
    y_ref[...] = jnp.dot(h, w_ref[...], preferred_element_type=jnp.float32)
    hr_ref[...] = (
        jnp.dot(h, root_ref[...], preferred_element_type=jnp.float32) + b_ref[...]
    )


def _final_body(p0_ref, p1_ref, hr_ref, o_ref):
    o_ref[...] = p0_ref[...] + p1_ref[...] + hr_ref[...]


_SC_MESH = plsc.VectorSubcoreMesh(core_axis_name="c", subcore_axis_name="s")


def _sc_body(y_hbm, idx_hbm, dst_hbm, ew_hbm, out_hbm,
             idx_v, dst_v, ew_v, rows_v, stage_v, acc_sh, gsem, ssem):
    c = lax.axis_index("c")
    s = lax.axis_index("s")
    wid = s * 2 + c

    def zero_loop(i, carry):
        stage_v[i, :] = jnp.zeros((H,), jnp.float32)
        return carry
    lax.fori_loop(0, NPT, zero_loop, 0)
    pltpu.sync_copy(stage_v, acc_sh.at[pl.ds(s * NPT, NPT)])
    plsc.subcore_barrier()

    base = wid * NCH
    pltpu.sync_copy(idx_hbm.at[pl.ds(base, NCH)], idx_v)
    pltpu.sync_copy(dst_hbm.at[pl.ds(base, NCH)], dst_v)
    pltpu.sync_copy(ew_hbm.at[pl.ds(base, NCH)], ew_v)

    def gfire(j, carry):
        pltpu.async_copy(y_hbm.at[idx_v.at[j]], rows_v.at[j], gsem)
        return carry
    lax.fori_loop(0, NCH, gfire, 0)

    def gdrain(j, carry):
        pltpu.make_async_copy(y_hbm.at[idx_v.at[j]], rows_v.at[j], gsem).wait()
        return carry
    lax.fori_loop(0, NCH, gdrain, 0)

    def scale_loop(j, carry):
        for i in range(CHUNK // 16):
            ewv = ew_v[j, pl.ds(i * 16, 16)]
            for l in range(16):
                e = i * 16 + l
                scale = jnp.broadcast_to(ewv[l], (16,))
                rows_v[j, e, :] = rows_v[j, e, :] * scale
        return carry
    lax.fori_loop(0, NCH, scale_loop, 0)

    def cfire(j, carry):
        pltpu.async_copy(rows_v.at[j], acc_sh.at[dst_v.at[j]], ssem, add=True)
        return carry
    lax.fori_loop(0, NCH, cfire, 0)

    def cdrain(j, carry):
        pltpu.make_async_copy(rows_v.at[j], acc_sh.at[dst_v.at[j]], ssem).wait()
        return carry
    lax.fori_loop(0, NCH, cdrain, 0)
    plsc.subcore_barrier()

    pltpu.sync_copy(acc_sh.at[pl.ds(s * NPT, NPT)], stage_v)
    pltpu.sync_copy(stage_v, out_hbm.at[c, pl.ds(s * NPT, NPT)])


_sc_pass = pl.kernel(
    _sc_body,
    out_type=jax.ShapeDtypeStruct((2, N, H), jnp.float32),
    mesh=_SC_MESH,
    scratch_types=[
        pltpu.VMEM((NCH, CHUNK), jnp.int32),
        pltpu.VMEM((NCH, CHUNK), jnp.int32),
        pltpu.VMEM((NCH, CHUNK), jnp.float32),
        pltpu.VMEM((NCH, CHUNK, H), jnp.float32),
        pltpu.VMEM((NPT, H), jnp.float32),
        pltpu.VMEM_SHARED((N, H), jnp.float32),
        pltpu.SemaphoreType.DMA,
        pltpu.SemaphoreType.DMA,
    ],
)


def kernel(x, edge_index, edge_attr, w1, root1, b1, w2, root2, b2):
    src2d = edge_index[0].reshape(E // 128, 128)
    et2d = edge_attr[:, 1].reshape(E // 128, 128)
    ew2d = edge_attr[:, 0].reshape(E // 128, 128)

    ewn2d, idx2d = pl.pallas_call(
        _edge_prep_body,
        out_shape=[
            jax.ShapeDtypeStruct((E // 128, 128), jnp.float32),
            jax.ShapeDtypeStruct((E // 128, 128), jnp.int32),
        ],
    )(ew2d, et2d, src2d)

    pad = EP - E
    idx_p = jnp.concatenate(
        [idx2d.reshape(E), jnp.zeros((pad,), jnp.int32)]).reshape(EP // 128, 128)
    dst_p = jnp.concatenate(
        [edge_index[1], jnp.zeros((pad,), jnp.int32)]).reshape(EP // 128, 128)
    ewn_p = jnp.concatenate(
        [ewn2d.reshape(E), jnp.zeros((pad,), jnp.float32)]).reshape(EP // 128, 128)

    w1cat = w1.transpose(1, 0, 2).reshape(F_IN, R * H)
    w2cat = w2.transpose(1, 0, 2).reshape(H, R * H)
    b1r = b1.reshape(1, H)
    b2r = b2.reshape(1, H)

    rep = lambda i: (0, 0)
    row = lambda i: (i, 0)

    y1, xr = pl.pallas_call(
        _dense1_body,
        grid=(N // BLK,),
        in_specs=[
            pl.BlockSpec((BLK, F_IN), row),
            pl.BlockSpec((F_IN, R * H), rep),
            pl.BlockSpec((F_IN, H), rep),
            pl.BlockSpec((1, H), rep),
        ],
        out_shape=[
            jax.ShapeDtypeStruct((N, R * H), jnp.float32),
            jax.ShapeDtypeStruct((N, H), jnp.float32),
        ],
        out_specs=[pl.BlockSpec((BLK, R * H), row), pl.BlockSpec((BLK, H), row)],
    )(x, w1cat, root1, b1r)

    part1 = _sc_pass(y1.reshape(N * R, H), idx_p, dst_p, ewn_p)

    y2, hr = pl.pallas_call(
        _mid_body,
        grid=(N // BLK,),
        in_specs=[
            pl.BlockSpec((BLK, H), row),
            pl.BlockSpec((BLK, H), row),
            pl.BlockSpec((BLK, H), row),
            pl.BlockSpec((H, R * H), rep),
            pl.BlockSpec((H, H), rep),
            pl.BlockSpec((1, H), rep),
        ],
        out_shape=[
            jax.ShapeDtypeStruct((N, R * H), jnp.float32),
            jax.ShapeDtypeStruct((N, H), jnp.float32),
        ],
        out_specs=[pl.BlockSpec((BLK, R * H), row), pl.BlockSpec((BLK, H), row)],
    )(part1[0], part1[1], xr, w2cat, root2, b2r)

    part2 = _sc_pass(y2.reshape(N * R, H), idx_p, dst_p, ewn_p)

    out = pl.pallas_call(
        _final_body,
        grid=(N // BLK,),
        in_specs=[
            pl.BlockSpec((BLK, H), row),
            pl.BlockSpec((BLK, H), row),
            pl.BlockSpec((BLK, H), row),
        ],
        out_shape=jax.ShapeDtypeStruct((N, H), jnp.float32),
        out_specs=pl.BlockSpec((BLK, H), row),
    )(part2[0], part2[1], hr)

    return out


# speedup vs baseline: 13.1928x; 13.1928x over previous
import functools

import jax
import jax.numpy as jnp
from jax import lax
from jax.experimental import pallas as pl
from jax.experimental.pallas import tpu as pltpu
from jax.experimental.pallas import tpu_sc as plsc

N = 10000
E = 160000
F_IN = 128
H = 16
R = 8

NW = 32
CHUNK = 128
NCH = 40
EPW = NCH * CHUNK
EP = NW * EPW
NPAD = 10240
NPT = NPAD // 16
BLK = 2000


def _edge_prep_body(ew_ref, et_ref, src_ref, ewn_ref, idx_ref):
    ew = ew_ref[...]
    mn = jnp.min(ew)
    mx = jnp.max(ew)
    inv = 1.0 / (mx - mn + 1e-8)
    ewn_ref[...] = (ew - mn) * inv
    idx_ref[...] = src_ref[...] * R + et_ref[...].astype(jnp.int32)


def _dense1_body(x_ref, w_ref, root_ref, b_ref, y_ref, xr_ref):
    x = x_ref[...]
    y_ref[...] = jnp.dot(x, w_ref[...], preferred_element_type=jnp.float32)
    xr_ref[...] = (
        jnp.dot(x, root_ref[...], preferred_element_type=jnp.float32) + b_ref[...]
    )


def _mid_body(p0_ref, p1_ref, xr_ref, w_ref, root_ref, b_ref, y_ref, hr_ref):
    h = jnp.maximum(p0_ref[...] + p1_ref[...] + xr_ref[...], 0.0)
    y_ref[...] = jnp.dot(h, w_ref[...], preferred_element_type=jnp.float32)
    hr_ref[...] = (
        jnp.dot(h, root_ref[...], preferred_element_type=jnp.float32) + b_ref[...]
    )


def _final_body(p0_ref, p1_ref, hr_ref, o_ref):
    o_ref[...] = p0_ref[...] + p1_ref[...] + hr_ref[...]


_SC_MESH = plsc.VectorSubcoreMesh(core_axis_name="c", subcore_axis_name="s")


def _sc_body(y_hbm, idx_hbm, dst_hbm, ew_hbm, out_hbm,
             idx_v, dst_v, ew_v, rows_v, stage_v, acc_sh, gsem, ssem):
    c = lax.axis_index("c")
    s = lax.axis_index("s")
    wid = s * 2 + c

    def zero_loop(i, carry):
        stage_v[i, :] = jnp.zeros((H,), jnp.float32)
        return carry
    lax.fori_loop(0, NPT, zero_loop, 0)
    pltpu.sync_copy(stage_v, acc_sh.at[pl.ds(s * NPT, NPT)])
    plsc.subcore_barrier()

    base = wid * NCH
    pltpu.sync_copy(idx_hbm.at[pl.ds(base, NCH)], idx_v)
    pltpu.sync_copy(dst_hbm.at[pl.ds(base, NCH)], dst_v)
    pltpu.sync_copy(ew_hbm.at[pl.ds(base, NCH)], ew_v)

    def gfire(j, carry):
        pltpu.async_copy(y_hbm.at[idx_v.at[j]], rows_v.at[j], gsem)
        return carry
    lax.fori_loop(0, NCH, gfire, 0)

    def gdrain(j, carry):
        pltpu.make_async_copy(y_hbm.at[idx_v.at[j]], rows_v.at[j], gsem).wait()
        return carry
    lax.fori_loop(0, NCH, gdrain, 0)

    def scale_loop(j, carry):
        for i in range(CHUNK // 16):
            ewv = ew_v[j, pl.ds(i * 16, 16)]
            for l in range(16):
                e = i * 16 + l
                scale = jnp.broadcast_to(ewv[l], (16,))
                rows_v[j, e, :] = rows_v[j, e, :] * scale
        return carry
    lax.fori_loop(0, NCH, scale_loop, 0)

    def cfire(j, carry):
        pltpu.async_copy(rows_v.at[j], acc_sh.at[dst_v.at[j]], ssem, add=True)
        return carry
    lax.fori_loop(0, NCH, cfire, 0)

    def cdrain(j, carry):
        pltpu.make_async_copy(rows_v.at[j], acc_sh.at[dst_v.at[j]], ssem).wait()
        return carry
    lax.fori_loop(0, NCH, cdrain, 0)
    plsc.subcore_barrier()

    pltpu.sync_copy(acc_sh.at[pl.ds(s * NPT, NPT)], stage_v)
    pltpu.sync_copy(stage_v, out_hbm.at[c, pl.ds(s * NPT, NPT)])


_sc_pass = pl.kernel(
    _sc_body,
    out_type=jax.ShapeDtypeStruct((2, NPAD, H), jnp.float32),
    mesh=_SC_MESH,
    scratch_types=[
        pltpu.VMEM((NCH, CHUNK), jnp.int32),
        pltpu.VMEM((NCH, CHUNK), jnp.int32),
        pltpu.VMEM((NCH, CHUNK), jnp.float32),
        pltpu.VMEM((NCH, CHUNK, H), jnp.float32),
        pltpu.VMEM((NPT, H), jnp.float32),
        pltpu.VMEM_SHARED((NPAD, H), jnp.float32),
        pltpu.SemaphoreType.DMA,
        pltpu.SemaphoreType.DMA,
    ],
    compiler_params=pltpu.CompilerParams(use_tc_tiling_on_sc=False),
)


def kernel(x, edge_index, edge_attr, w1, root1, b1, w2, root2, b2):
    src2d = edge_index[0].reshape(E // 128, 128)
    et2d = edge_attr[:, 1].reshape(E // 128, 128)
    ew2d = edge_attr[:, 0].reshape(E // 128, 128)

    ewn2d, idx2d = pl.pallas_call(
        _edge_prep_body,
        out_shape=[
            jax.ShapeDtypeStruct((E // 128, 128), jnp.float32),
            jax.ShapeDtypeStruct((E // 128, 128), jnp.int32),
        ],
    )(ew2d, et2d, src2d)

    pad = EP - E
    idx_p = jnp.concatenate(
        [idx2d.reshape(E), jnp.zeros((pad,), jnp.int32)]).reshape(EP // 128, 128)
    dst_p = jnp.concatenate(
        [edge_index[1], jnp.zeros((pad,), jnp.int32)]).reshape(EP // 128, 128)
    ewn_p = jnp.concatenate(
        [ewn2d.reshape(E), jnp.zeros((pad,), jnp.float32)]).reshape(EP // 128, 128)

    w1cat = w1.transpose(1, 0, 2).reshape(F_IN, R * H)
    w2cat = w2.transpose(1, 0, 2).reshape(H, R * H)
    b1r = b1.reshape(1, H)
    b2r = b2.reshape(1, H)

    rep = lambda i: (0, 0)
    row = lambda i: (i, 0)

    y1, xr = pl.pallas_call(
        _dense1_body,
        grid=(N // BLK,),
        in_specs=[
            pl.BlockSpec((BLK, F_IN), row),
            pl.BlockSpec((F_IN, R * H), rep),
            pl.BlockSpec((F_IN, H), rep),
            pl.BlockSpec((1, H), rep),
        ],
        out_shape=[
            jax.ShapeDtypeStruct((N, R * H), jnp.float32),
            jax.ShapeDtypeStruct((N, H), jnp.float32),
        ],
        out_specs=[pl.BlockSpec((BLK, R * H), row), pl.BlockSpec((BLK, H), row)],
    )(x, w1cat, root1, b1r)

    part1 = _sc_pass(y1.reshape(N * R, H), idx_p, dst_p, ewn_p)

    y2, hr = pl.pallas_call(
        _mid_body,
        grid=(N // BLK,),
        in_specs=[
            pl.BlockSpec((BLK, H), row),
            pl.BlockSpec((BLK, H), row),
            pl.BlockSpec((BLK, H), row),
            pl.BlockSpec((H, R * H), rep),
            pl.BlockSpec((H, H), rep),
            pl.BlockSpec((1, H), rep),
        ],
        out_shape=[
            jax.ShapeDtypeStruct((N, R * H), jnp.float32),
            jax.ShapeDtypeStruct((N, H), jnp.float32),
        ],
        out_specs=[pl.BlockSpec((BLK, R * H), row), pl.BlockSpec((BLK, H), row)],
    )(part1[0, :N], part1[1, :N], xr, w2cat, root2, b2r)

    part2 = _sc_pass(y2.reshape(N * R, H), idx_p, dst_p, ewn_p)

    out = pl.pallas_call(
        _final_body,
        grid=(N // BLK,),
        in_specs=[
            pl.BlockSpec((BLK, H), row),
            pl.BlockSpec((BLK, H), row),
            pl.BlockSpec((BLK, H), row),
        ],
        out_shape=jax.ShapeDtypeStruct((N, H), jnp.float32),
        out_specs=pl.BlockSpec((BLK, H), row),
    )(part2[0, :N], part2[1, :N], hr)

    return out
